# baseline (speedup 1.0000x reference)
import jax
import jax.numpy as jnp
from jax.experimental import pallas as pl
from jax.experimental.pallas import tpu as pltpu

_LATENTS = (32, 64, 96, 128, 192, 256)
_NH = 6
_DMAX = 256
_TILE = 512


def _prep_body(*refs):
    pr = refs[:60]
    (w1e_o, w1c_o, w1n_o, b1a_o, w3c_o, bc_o, lnb_o, aux_o) = refs[60:]
    for i in range(_NH):
        (w1_r, b1_r, a1_r, _w2, _b2, _a2, w3_r, b3_r, lnw_r, lnb_r) = \
            pr[10 * i:10 * (i + 1)]
        d = _LATENTS[i]
        s = slice(i * 128, (i + 1) * 128)
        w1 = w1_r[...]
        w1e_o[s, :] = w1[:, 0:8]
        w1c_o[s, :] = w1[:, 8:72]
        w1n_o[s, :] = w1[:, 72:73]
        b1a_o[0:1, s] = b1_r[...][None, :]
        b1a_o[1:2, s] = jnp.full((1, 128), a1_r[0], jnp.float32)
        w3 = w3_r[...]
        wbar = jnp.mean(w3, axis=0, keepdims=True)
        w3p = w3 - wbar
        lnw = lnw_r[...]
        b3 = b3_r[...]
        b3p = b3 - jnp.mean(b3)
        sc = w3p * lnw[:, None]
        if d < _DMAX:
            z2 = jnp.zeros((_DMAX - d, 64), jnp.float32)
            zp = jnp.zeros((_DMAX - d,), jnp.float32)
            w3c_o[i] = jnp.concatenate([sc, z2, w3p, z2], axis=0)
            bc_o[i, :] = jnp.concatenate([lnw * b3p, zp, b3p, zp])
            lnb_o[i, :] = jnp.concatenate([lnb_r[...], zp])
        else:
            w3c_o[i] = jnp.concatenate([sc, w3p], axis=0)
            bc_o[i, :] = jnp.concatenate([lnw * b3p, b3p])
            lnb_o[i, :] = lnb_r[...]
        aux_o[i, :] = jnp.full((8,), 1.0 / d, jnp.float32)


def _main_body(eq_ref, csi_ref, np_ref, r_ref, w1e_ref, w1c_ref, w1n_ref,
               b1a_ref, w3c_ref, bc_ref, lnb_ref, aux_ref, *refs):
    o_ref = refs[-1]
    heads = refs[:-1]

    def dot1(a, b):
        return jax.lax.dot_general(a, b, (((1,), (1,)), ((), ())),
                                   preferred_element_type=jnp.float32)

    def dot0(a, b):
        return jax.lax.dot_general(a, b, (((1,), (0,)), ((), ())),
                                   preferred_element_type=jnp.float32)

    r = r_ref[...]
    best = jnp.max(r, axis=1, keepdims=True)
    eqm = (r == best).astype(jnp.float32)
    lt = jnp.tril(jnp.ones((_NH, _NH), jnp.float32))
    cums = dot0(eqm, lt)
    fm = eqm * (cums == 1.0).astype(jnp.float32)

    h1 = dot1(eq_ref[...], w1e_ref[...]) + dot1(csi_ref[...], w1c_ref[...]) \
        + dot1(np_ref[...], w1n_ref[...]) + b1a_ref[0:1, :]
    a1v = b1a_ref[1:2, :]
    h1 = jnp.where(h1 >= 0, h1, a1v * h1)

    acc = jnp.zeros((h1.shape[0], 2 * _DMAX), jnp.float32)
    for i in range(_NH):
        w2_r, b2_r, a2_r = heads[3 * i:3 * i + 3]
        h = h1[:, i * 128:(i + 1) * 128]
        h2 = dot1(h, w2_r[...]) + b2_r[...]
        h2 = jnp.where(h2 >= 0, h2, a2_r[0] * h2)
        h2 = h2 * fm[:, i:i + 1]
        acc = acc + dot1(h2, w3c_ref[i])
    acc = acc + dot0(fm, bc_ref[...])
    z = acc[:, :_DMAX]
    u = acc[:, _DMAX:]
    ss = dot1(u * u, jnp.ones((1, _DMAX), jnp.float32))
    invd = dot0(fm, aux_ref[...])
    lnb = dot0(fm, lnb_ref[...])
    rs = jax.lax.rsqrt(ss * invd[:, 0:1] + 1e-5)
    o_ref[...] = z * rs + lnb


def kernel(equalized_symbol, csi_context, noise_power, rate_one_hot, params):
    b = equalized_symbol.shape[0]
    f32 = jnp.float32
    smem = pl.BlockSpec(memory_space=pltpu.SMEM)
    fullb = lambda a: pl.BlockSpec(a.shape, lambda i: (0,) * a.ndim)

    pargs, pspecs = [], []
    for p in params:
        for k in ('W1', 'b1', 'a1', 'W2', 'b2', 'a2', 'W3', 'b3', 'ln_w', 'ln_b'):
            v = p[k]
            pargs.append(v)
            pspecs.append(smem if k in ('a1', 'a2') else
                          pl.BlockSpec(v.shape, lambda n=v.ndim: (0,) * n))

    prep = pl.pallas_call(
        _prep_body,
        in_specs=pspecs,
        out_specs=[pl.BlockSpec(s, lambda n=len(s): (0,) * n) for s in
                   [(768, 8), (768, 64), (768, 1), (2, 768),
                    (_NH, 2 * _DMAX, 64), (_NH, 2 * _DMAX), (_NH, _DMAX),
                    (_NH, 8)]],
        out_shape=[jax.ShapeDtypeStruct(s, f32) for s in
                   [(768, 8), (768, 64), (768, 1), (2, 768),
                    (_NH, 2 * _DMAX, 64), (_NH, 2 * _DMAX), (_NH, _DMAX),
                    (_NH, 8)]],
    )(*pargs)
    w1e, w1c, w1n, b1a, w3c, bc, lnbt, aux = prep

    hargs, hspecs = [], []
    for p in params:
        hargs += [p['W2'], p['b2'], p['a2']]
        hspecs += [fullb(p['W2']), fullb(p['b2']), smem]

    grid = (b // _TILE,)
    tb = lambda w: pl.BlockSpec((_TILE, w), lambda i: (i, 0))
    out = pl.pallas_call(
        _main_body,
        grid=grid,
        in_specs=[tb(8), tb(64), tb(1), tb(_NH),
                  fullb(w1e), fullb(w1c), fullb(w1n), fullb(b1a),
                  fullb(w3c), fullb(bc), fullb(lnbt), fullb(aux),
                  *hspecs],
        out_specs=pl.BlockSpec((_TILE, _DMAX), lambda i: (i, 0)),
        out_shape=jax.ShapeDtypeStruct((b, _DMAX), f32),
    )(equalized_symbol, csi_context, noise_power[:, None], rate_one_hot,
      w1e, w1c, w1n, b1a, w3c, bc, lnbt, aux, *hargs)
    return out


# speedup vs baseline: 1.0002x; 1.0002x over previous
import jax
import jax.numpy as jnp
from jax.experimental import pallas as pl
from jax.experimental.pallas import tpu as pltpu

_LATENTS = (32, 64, 96, 128, 192, 256)
_NH = 6
_DMAX = 256
_TILE = 512


def _prep_body(*refs):
    pr = refs[:60]
    (w1e_o, w1c_o, w1n_o, b1a_o, w3c_o, bc_o, lnb_o, aux_o) = refs[60:]
    for i in range(_NH):
        (w1_r, b1_r, a1_r, _w2, _b2, _a2, w3_r, b3_r, lnw_r, lnb_r) = \
            pr[10 * i:10 * (i + 1)]
        d = _LATENTS[i]
        s = slice(i * 128, (i + 1) * 128)
        w1 = w1_r[...]
        w1e_o[s, :] = w1[:, 0:8]
        w1c_o[s, :] = w1[:, 8:72]
        w1n_o[s, :] = w1[:, 72:73]
        b1a_o[0:1, s] = b1_r[...][None, :]
        b1a_o[1:2, s] = jnp.full((1, 128), a1_r[0], jnp.float32)
        w3 = w3_r[...]
        wbar = jnp.mean(w3, axis=0, keepdims=True)
        w3p = w3 - wbar
        lnw = lnw_r[...]
        b3 = b3_r[...]
        b3p = b3 - jnp.mean(b3)
        sc = w3p * lnw[:, None]
        if d < _DMAX:
            z2 = jnp.zeros((_DMAX - d, 64), jnp.float32)
            zp = jnp.zeros((_DMAX - d,), jnp.float32)
            w3c_o[i] = jnp.concatenate([sc, z2, w3p, z2], axis=0).T
            bc_o[i, :] = jnp.concatenate([lnw * b3p, zp, b3p, zp])
            lnb_o[i, :] = jnp.concatenate([lnb_r[...], zp])
        else:
            w3c_o[i] = jnp.concatenate([sc, w3p], axis=0).T
            bc_o[i, :] = jnp.concatenate([lnw * b3p, b3p])
            lnb_o[i, :] = lnb_r[...]
        aux_o[i, :] = jnp.full((8,), 1.0 / d, jnp.float32)


def _main_body(eq_ref, csi_ref, np_ref, r_ref, w1e_ref, w1c_ref, w1n_ref,
               b1a_ref, w3c_ref, bc_ref, lnb_ref, aux_ref, *refs):
    o_ref = refs[-1]
    heads = refs[:-1]

    def dot1(a, b):
        return jax.lax.dot_general(a, b, (((1,), (1,)), ((), ())),
                                   preferred_element_type=jnp.float32)

    def dot0(a, b):
        return jax.lax.dot_general(a, b, (((1,), (0,)), ((), ())),
                                   preferred_element_type=jnp.float32)

    r = r_ref[...]
    best = jnp.max(r, axis=1, keepdims=True)
    eqm = (r == best).astype(jnp.float32)
    lt = jnp.tril(jnp.ones((_NH, _NH), jnp.float32))
    cums = dot0(eqm, lt)
    fm = eqm * (cums == 1.0).astype(jnp.float32)

    h1 = dot1(eq_ref[...], w1e_ref[...]) + dot1(csi_ref[...], w1c_ref[...]) \
        + dot1(np_ref[...], w1n_ref[...]) + b1a_ref[0:1, :]
    a1v = b1a_ref[1:2, :]
    h1 = jnp.where(h1 >= 0, h1, a1v * h1)

    acc = jnp.zeros((h1.shape[0], 2 * _DMAX), jnp.float32)
    for i in range(_NH):
        w2_r, b2_r, a2_r = heads[3 * i:3 * i + 3]
        h = h1[:, i * 128:(i + 1) * 128]
        h2 = dot1(h, w2_r[...]) + b2_r[...]
        h2 = jnp.where(h2 >= 0, h2, a2_r[0] * h2)
        h2 = h2 * fm[:, i:i + 1]
        acc = acc + dot0(h2, w3c_ref[i])
    acc = acc + dot0(fm, bc_ref[...])
    z = acc[:, :_DMAX]
    u = acc[:, _DMAX:]
    ss = dot1(u * u, jnp.ones((1, _DMAX), jnp.float32))
    invd = dot0(fm, aux_ref[...])
    lnb = dot0(fm, lnb_ref[...])
    rs = jax.lax.rsqrt(ss * invd[:, 0:1] + 1e-5)
    o_ref[...] = z * rs + lnb


def kernel(equalized_symbol, csi_context, noise_power, rate_one_hot, params):
    b = equalized_symbol.shape[0]
    f32 = jnp.float32
    smem = pl.BlockSpec(memory_space=pltpu.SMEM)
    fullb = lambda a: pl.BlockSpec(a.shape, lambda i: (0,) * a.ndim)

    pargs, pspecs = [], []
    for p in params:
        for k in ('W1', 'b1', 'a1', 'W2', 'b2', 'a2', 'W3', 'b3', 'ln_w', 'ln_b'):
            v = p[k]
            pargs.append(v)
            pspecs.append(smem if k in ('a1', 'a2') else
                          pl.BlockSpec(v.shape, lambda n=v.ndim: (0,) * n))

    prep = pl.pallas_call(
        _prep_body,
        in_specs=pspecs,
        out_specs=[pl.BlockSpec(s, lambda n=len(s): (0,) * n) for s in
                   [(768, 8), (768, 64), (768, 1), (2, 768),
                    (_NH, 64, 2 * _DMAX), (_NH, 2 * _DMAX), (_NH, _DMAX),
                    (_NH, 8)]],
        out_shape=[jax.ShapeDtypeStruct(s, f32) for s in
                   [(768, 8), (768, 64), (768, 1), (2, 768),
                    (_NH, 64, 2 * _DMAX), (_NH, 2 * _DMAX), (_NH, _DMAX),
                    (_NH, 8)]],
    )(*pargs)
    w1e, w1c, w1n, b1a, w3c, bc, lnbt, aux = prep

    hargs, hspecs = [], []
    for p in params:
        hargs += [p['W2'], p['b2'], p['a2']]
        hspecs += [fullb(p['W2']), fullb(p['b2']), smem]

    grid = (b // _TILE,)
    tb = lambda w: pl.BlockSpec((_TILE, w), lambda i: (i, 0))
    out = pl.pallas_call(
        _main_body,
        grid=grid,
        in_specs=[tb(8), tb(64), tb(1), tb(_NH),
                  fullb(w1e), fullb(w1c), fullb(w1n), fullb(b1a),
                  fullb(w3c), fullb(bc), fullb(lnbt), fullb(aux),
                  *hspecs],
        out_specs=pl.BlockSpec((_TILE, _DMAX), lambda i: (i, 0)),
        out_shape=jax.ShapeDtypeStruct((b, _DMAX), f32),
    )(equalized_symbol, csi_context, noise_power[:, None], rate_one_hot,
      w1e, w1c, w1n, b1a, w3c, bc, lnbt, aux, *hargs)
    return out


# speedup vs baseline: 1.1943x; 1.1941x over previous
import jax
import jax.numpy as jnp
from jax.experimental import pallas as pl
from jax.experimental.pallas import tpu as pltpu

_LATENTS = (32, 64, 96, 128, 192, 256)
_NH = 6
_DIN = 73
_DP = 128
_DMAX = 256
_TILE = 512
_BATCH = 16384


def _fused_body(x_ref, r_ref, w1_ref, b1_ref, a_ref, w2_ref, b2_ref,
                w3_ref, b3_ref, lnw_ref, lnb_ref, o_ref):
    x = x_ref[...]
    best = r_ref[:, 0:1]
    e = jnp.zeros((x.shape[0], 1), jnp.int32)
    for j in range(1, _NH):
        rj = r_ref[:, j:j + 1]
        m = rj > best
        e = jnp.where(m, j, e)
        best = jnp.maximum(best, rj)

    acc = jnp.zeros((x.shape[0], _DMAX), jnp.float32)
    for i in range(_NH):
        d = _LATENTS[i]
        h = jax.lax.dot_general(x, w1_ref[i], (((1,), (1,)), ((), ())),
                                preferred_element_type=jnp.float32)
        h = h + b1_ref[i]
        a1 = a_ref[i, 0]
        h = jnp.where(h >= 0, h, a1 * h)
        h = jax.lax.dot_general(h, w2_ref[i], (((1,), (1,)), ((), ())),
                                preferred_element_type=jnp.float32)
        h = h + b2_ref[i]
        a2 = a_ref[i, 1]
        h = jnp.where(h >= 0, h, a2 * h)
        h = jax.lax.dot_general(h, w3_ref[i], (((1,), (1,)), ((), ())),
                                preferred_element_type=jnp.float32)
        h = h + b3_ref[i]
        mu = jnp.sum(h, axis=1, keepdims=True) * (1.0 / d)
        col = jax.lax.broadcasted_iota(jnp.int32, h.shape, 1)
        diff = jnp.where(col < d, h - mu, 0.0)
        var = jnp.sum(diff * diff, axis=1, keepdims=True) * (1.0 / d)
        y = diff * jax.lax.rsqrt(var + 1e-5) * lnw_ref[i] + lnb_ref[i]
        mask = (e == i).astype(jnp.float32)
        acc = acc + mask * y
    o_ref[...] = acc


def kernel(equalized_symbol, csi_context, noise_power, rate_one_hot, params):
    b = equalized_symbol.shape[0]
    combined = jnp.concatenate(
        [equalized_symbol, csi_context, noise_power[:, None],
         jnp.zeros((b, _DP - _DIN), jnp.float32)], axis=1)

    w1s = jnp.stack([jnp.pad(p['W1'], ((0, 0), (0, _DP - _DIN))) for p in params])
    b1s = jnp.stack([p['b1'][None, :] for p in params])
    w2s = jnp.stack([p['W2'] for p in params])
    b2s = jnp.stack([p['b2'][None, :] for p in params])
    w3s = jnp.stack([jnp.pad(p['W3'], ((0, _DMAX - p['W3'].shape[0]), (0, 0)))
                     for p in params])
    b3s = jnp.stack([jnp.pad(p['b3'], (0, _DMAX - p['b3'].shape[0]))[None, :]
                     for p in params])
    lnws = jnp.stack([jnp.pad(p['ln_w'], (0, _DMAX - p['ln_w'].shape[0]))[None, :]
                      for p in params])
    lnbs = jnp.stack([jnp.pad(p['ln_b'], (0, _DMAX - p['ln_b'].shape[0]))[None, :]
                      for p in params])
    a_all = jnp.stack([jnp.concatenate([p['a1'], p['a2']]) for p in params])

    grid = (b // _TILE,)
    full = lambda shp: pl.BlockSpec(shp, lambda i: (0,) * len(shp))
    out = pl.pallas_call(
        _fused_body,
        grid=grid,
        in_specs=[
            pl.BlockSpec((_TILE, _DP), lambda i: (i, 0)),
            pl.BlockSpec((_TILE, _NH), lambda i: (i, 0)),
            full((_NH, _DP, _DP)),
            full((_NH, 1, _DP)),
            pl.BlockSpec(memory_space=pltpu.SMEM),
            full((_NH, 64, _DP)),
            full((_NH, 1, 64)),
            full((_NH, _DMAX, 64)),
            full((_NH, 1, _DMAX)),
            full((_NH, 1, _DMAX)),
            full((_NH, 1, _DMAX)),
        ],
        out_specs=pl.BlockSpec((_TILE, _DMAX), lambda i: (i, 0)),
        out_shape=jax.ShapeDtypeStruct((b, _DMAX), jnp.float32),
    )(combined, rate_one_hot, w1s, b1s, a_all, w2s, b2s, w3s, b3s, lnws, lnbs)
    return out
